# baseline (speedup 1.0000x reference)
import functools

import jax
import jax.numpy as jnp
from jax.experimental import pallas as pl
from jax.experimental.pallas import tpu as pltpu

ALPHA = 0.8
GAMMA = 0.8

LANES = 512
AGENTS_PER_ROW = 128
BLOCK_ROWS = 256


def _expand4(x, e):
    return jax.lax.dot_general(
        x, e, (((1,), (0,)), ((), ())), preferred_element_type=jnp.float32
    )


def _q_update_kernel(a_ref, b_ref, p_ref, q_ref, o_ref):
    q = q_ref[...]

    ci = jax.lax.broadcasted_iota(jnp.int32, (AGENTS_PER_ROW, LANES), 1)
    ri = jax.lax.broadcasted_iota(jnp.int32, (AGENTS_PER_ROW, LANES), 0)
    e = ((ci >> 2) == ri).astype(jnp.bfloat16)

    s = (2 * a_ref[...] + b_ref[...]).astype(jnp.bfloat16)
    p = p_ref[...]
    p_hi = p.astype(jnp.bfloat16)
    p_lo = (p - p_hi.astype(jnp.float32)).astype(jnp.bfloat16)

    s4 = _expand4(s, e)
    p4 = _expand4(p_hi, e) + _expand4(p_lo, e)
    s4i = s4.astype(jnp.int32)
    b4 = s4i & 1

    lane = jax.lax.broadcasted_iota(jnp.int32, q.shape, 1)
    r = lane & 3
    even = (lane & 1) == 0
    low = (lane & 2) == 0

    partner = jnp.where(even, pltpu.roll(q, LANES - 1, 1), pltpu.roll(q, 1, 1))
    pm = jnp.maximum(q, partner)
    pm_other = jnp.where(low, pltpu.roll(pm, LANES - 2, 1), pltpu.roll(pm, 2, 1))
    m_low = jnp.where(low, pm, pm_other)
    m_high = jnp.where(low, pm_other, pm)
    m = jnp.where(b4 == 1, m_high, m_low)

    sel = s4i == r
    upd = q + ALPHA * (p4 + GAMMA * m - q)
    o_ref[...] = jnp.where(sel, upd, q)


@jax.jit
def kernel(type_t_matrix, type_t1_matrix, Q_tensor, profit_matrix):
    n = type_t_matrix.size
    rows = n // AGENTS_PER_ROW
    a = type_t_matrix.reshape(rows, AGENTS_PER_ROW).astype(jnp.int32)
    b = type_t1_matrix.reshape(rows, AGENTS_PER_ROW).astype(jnp.int32)
    p = profit_matrix.reshape(rows, AGENTS_PER_ROW).astype(jnp.float32)
    q = Q_tensor.reshape(rows, LANES)

    grid = rows // BLOCK_ROWS
    agent_spec = pl.BlockSpec((BLOCK_ROWS, AGENTS_PER_ROW), lambda i: (i, 0))
    q_spec = pl.BlockSpec((BLOCK_ROWS, LANES), lambda i: (i, 0))

    out = pl.pallas_call(
        _q_update_kernel,
        grid=(grid,),
        in_specs=[agent_spec, agent_spec, agent_spec, q_spec],
        out_specs=q_spec,
        out_shape=jax.ShapeDtypeStruct((rows, LANES), jnp.float32),
        compiler_params=pltpu.CompilerParams(
            dimension_semantics=("arbitrary",),
        ),
    )(a, b, p, q)
    return out.reshape(Q_tensor.shape)


# speedup vs baseline: 233.2361x; 233.2361x over previous
import jax
import jax.numpy as jnp
from jax.experimental import pallas as pl
from jax.experimental.pallas import tpu as pltpu

ALPHA = 0.8
GAMMA = 0.8

BLK = 64 * 2048


def _q_update_kernel(q_ref, a_ref, b_ref, p_ref, o_ref):
    q00 = q_ref[0, 0]
    q01 = q_ref[0, 1]
    q10 = q_ref[1, 0]
    q11 = q_ref[1, 1]
    a = a_ref[...]
    b = b_ref[...]
    p = p_ref[...]

    b0 = b == 0
    m = jnp.where(b0, jnp.maximum(q00, q01), jnp.maximum(q10, q11))
    old = jnp.where(
        a == 0, jnp.where(b0, q00, q01), jnp.where(b0, q10, q11)
    )
    u = old + ALPHA * (p + GAMMA * m - old)

    a0 = a == 0
    o_ref[0, 0] = jnp.where(a0 & b0, u, q00)
    o_ref[0, 1] = jnp.where(a0 & ~b0, u, q01)
    o_ref[1, 0] = jnp.where(~a0 & b0, u, q10)
    o_ref[1, 1] = jnp.where(~a0 & ~b0, u, q11)


@jax.jit
def kernel(type_t_matrix, type_t1_matrix, Q_tensor, profit_matrix):
    n = type_t_matrix.size
    a = type_t_matrix.reshape(n).astype(jnp.int32)
    b = type_t1_matrix.reshape(n).astype(jnp.int32)
    p = profit_matrix.reshape(n).astype(jnp.float32)
    qt = jnp.transpose(Q_tensor, (1, 2, 0))

    grid = n // BLK
    q_spec = pl.BlockSpec((2, 2, BLK), lambda i: (0, 0, i))
    v_spec = pl.BlockSpec((BLK,), lambda i: (i,))

    out = pl.pallas_call(
        _q_update_kernel,
        grid=(grid,),
        in_specs=[q_spec, v_spec, v_spec, v_spec],
        out_specs=q_spec,
        out_shape=jax.ShapeDtypeStruct((2, 2, n), jnp.float32),
        compiler_params=pltpu.CompilerParams(
            dimension_semantics=("arbitrary",),
        ),
    )(qt, a, b, p)
    return jnp.transpose(out, (2, 0, 1))
